# baseline (speedup 1.0000x reference)
import functools

import jax
import jax.numpy as jnp
from jax import lax
from jax.experimental import pallas as pl
from jax.experimental.pallas import tpu as pltpu
from jax.experimental.pallas import tpu_sc as plsc

N_NODES = 10000
IN_DIM = 128
OUT_DIM = 128
NUM_RELATIONS = 10
N_EDGES = 320000

N_REL1 = NUM_RELATIONS + 1
HALF = OUT_DIM // 2

NC = 2
NS = 16
NW = NC * NS
CHUNK = 128
NDMA = -(-(-(-N_EDGES // NW)) // CHUNK)
NCH = NDMA
EPW = NDMA * CHUNK
E_PAD = NW * EPW
ACC_ROWS = 10112
SPW = ACC_ROWS // NS
LANES = 16
SCALE = 256.0


_BM = 10000


def _transform_body(x_ref, w_ref, b_ref, t_ref):
    res = (
        jnp.dot(x_ref[...], w_ref[0], preferred_element_type=jnp.float32)
        + b_ref[0]
    )
    t_ref[0] = jnp.floor(res * SCALE + 0.5).astype(jnp.int16)


def _transform(x, w_all, b_all):
    return pl.pallas_call(
        _transform_body,
        grid=(N_NODES // _BM, N_REL1),
        in_specs=[
            pl.BlockSpec((_BM, IN_DIM), lambda j, r: (j, 0)),
            pl.BlockSpec((1, IN_DIM, OUT_DIM), lambda j, r: (r, 0, 0)),
            pl.BlockSpec((1, 1, OUT_DIM), lambda j, r: (r, 0, 0)),
        ],
        out_specs=pl.BlockSpec((1, _BM, OUT_DIM), lambda j, r: (r, j, 0)),
        out_shape=jax.ShapeDtypeStruct((N_REL1, N_NODES, OUT_DIM), jnp.int16),
    )(x, w_all, b_all)


_MESH = plsc.VectorSubcoreMesh(
    core_axis_name="c", subcore_axis_name="s", num_cores=NC, num_subcores=NS
)


@functools.partial(
    pl.kernel,
    out_type=jax.ShapeDtypeStruct((NC, ACC_ROWS, OUT_DIM), jnp.int16),
    mesh=_MESH,
    scratch_types=[
        pltpu.VMEM((NCH, CHUNK), jnp.int32),
        pltpu.VMEM((NCH, CHUNK), jnp.int32),
        pltpu.VMEM((CHUNK, OUT_DIM), jnp.int16),
        pltpu.VMEM((CHUNK, OUT_DIM), jnp.int16),
        pltpu.VMEM_SHARED((ACC_ROWS, OUT_DIM), jnp.int16),
        pltpu.SemaphoreType.DMA,
        pltpu.SemaphoreType.DMA,
    ],
    compiler_params=pltpu.CompilerParams(use_tc_tiling_on_sc=False),
)
def _edge_scatter(tbl_hbm, gidx_hbm, dst_hbm, out_hbm,
                  gidx_v, dst_v, rows_a, rows_b, acc_sh, sem_a, sem_b):
    c = lax.axis_index("c")
    s = lax.axis_index("s")

    w = c * NS + s

    @pl.loop(0, CHUNK)
    def _zrow(r):
        for u in range(OUT_DIM // (2 * LANES)):
            rows_a[r, pl.ds(u * 2 * LANES, 2 * LANES)] = jnp.zeros(
                (2 * LANES,), jnp.int16
            )

    for i in range(SPW // CHUNK):
        pltpu.sync_copy(rows_a, acc_sh.at[pl.ds(s * SPW + i * CHUNK, CHUNK)])
    _rem = SPW % CHUNK
    if _rem:
        pltpu.sync_copy(
            rows_a.at[pl.ds(0, _rem)],
            acc_sh.at[pl.ds(s * SPW + (SPW // CHUNK) * CHUNK, _rem)],
        )
    pltpu.sync_copy(gidx_hbm.at[w], gidx_v)
    pltpu.sync_copy(dst_hbm.at[w], dst_v)
    plsc.subcore_barrier()

    def _gi(m):
        return gidx_v.at[m]

    def _di(m):
        return dst_v.at[m]

    pltpu.async_copy(tbl_hbm.at[_gi(0)], rows_a, sem_a)

    @pl.loop(0, (NDMA - 1) // 2)
    def _pairs(k):
        m = 2 * k
        pltpu.async_copy(tbl_hbm.at[_gi(m + 1)], rows_b, sem_b)
        pltpu.make_async_copy(tbl_hbm.at[_gi(m)], rows_a, sem_a).wait()
        pltpu.sync_copy(rows_a, acc_sh.at[_di(m)], add=True)
        pltpu.async_copy(tbl_hbm.at[_gi(m + 2)], rows_a, sem_a)
        pltpu.make_async_copy(tbl_hbm.at[_gi(m + 1)], rows_b, sem_b).wait()
        pltpu.sync_copy(rows_b, acc_sh.at[_di(m + 1)], add=True)

    pltpu.make_async_copy(tbl_hbm.at[_gi(NDMA - 1)], rows_a, sem_a).wait()
    pltpu.sync_copy(rows_a, acc_sh.at[_di(NDMA - 1)], add=True)

    plsc.subcore_barrier()
    pltpu.sync_copy(
        acc_sh.at[pl.ds(s * SPW, SPW)], out_hbm.at[c, pl.ds(s * SPW, SPW)]
    )


_BL = 10000


def _ln_body(t_ref, p_ref, w_ref, b_ref, o_ref):
    agg = (
        t_ref[0].astype(jnp.int32)
        + p_ref[0].astype(jnp.int32)
        + p_ref[1].astype(jnp.int32)
    )
    total = agg.astype(jnp.float32) * (1.0 / SCALE)
    mu = jnp.mean(total, axis=1, keepdims=True)
    d = total - mu
    var = jnp.mean(d * d, axis=1, keepdims=True)
    o_ref[...] = d * lax.rsqrt(var + 1e-5) * w_ref[0] + b_ref[0]


def _layernorm(t, partials, ln_w, ln_b):
    return pl.pallas_call(
        _ln_body,
        grid=(N_NODES // _BL,),
        in_specs=[
            pl.BlockSpec((1, _BL, OUT_DIM), lambda j: (N_REL1 - 1, j, 0)),
            pl.BlockSpec((NC, _BL, OUT_DIM), lambda j: (0, j, 0)),
            pl.BlockSpec((1, OUT_DIM), lambda j: (0, 0)),
            pl.BlockSpec((1, OUT_DIM), lambda j: (0, 0)),
        ],
        out_specs=pl.BlockSpec((_BL, OUT_DIM), lambda j: (j, 0)),
        out_shape=jax.ShapeDtypeStruct((N_NODES, OUT_DIM), jnp.float32),
    )(t, partials, ln_w, ln_b)


def kernel(x, edge_index, edge_type, relation_weights, relation_bias,
           self_loop_w, self_loop_b, ln_weight, ln_bias):
    x = x.astype(jnp.float32)
    src = edge_index[0].astype(jnp.int32)
    dst = edge_index[1].astype(jnp.int32)
    et = edge_type.astype(jnp.int32)

    w_all = jnp.concatenate([relation_weights, self_loop_w.T[None]], axis=0)
    b_all = jnp.concatenate(
        [relation_bias, self_loop_b[None]], axis=0
    ).reshape(N_REL1, 1, OUT_DIM)

    t = _transform(x, w_all, b_all)
    tbl = t.reshape(N_REL1 * N_NODES, OUT_DIM)

    pad = E_PAD - N_EDGES
    gidx = jnp.concatenate(
        [et * N_NODES + src, jnp.zeros((pad,), jnp.int32)]
    ).reshape(NW, NCH, CHUNK)
    dstp = jnp.concatenate(
        [dst, jnp.full((pad,), N_NODES, jnp.int32)]
    ).reshape(NW, NCH, CHUNK)

    partials = _edge_scatter(tbl, gidx, dstp)

    return _layernorm(
        t, partials, ln_weight.reshape(1, OUT_DIM), ln_bias.reshape(1, OUT_DIM)
    )


# speedup vs baseline: 1.2763x; 1.2763x over previous
import functools

import jax
import jax.numpy as jnp
from jax import lax
from jax.experimental import pallas as pl
from jax.experimental.pallas import tpu as pltpu
from jax.experimental.pallas import tpu_sc as plsc

N_NODES = 10000
IN_DIM = 128
OUT_DIM = 128
NUM_RELATIONS = 10
N_EDGES = 320000

N_REL1 = NUM_RELATIONS + 1
HALF = OUT_DIM // 2

NC = 2
NS = 16
CHUNK = 128
NDMA = -(-(-(-N_EDGES // NS)) // CHUNK)
NCH = NDMA
EPW = NDMA * CHUNK
E_PAD = NS * EPW
ACC_ROWS = 10112
SPW = ACC_ROWS // NS
LANES = 16


_BM = 10000


def _transform_body(x_ref, w_ref, b_ref, t_ref):
    t_ref[0] = (
        jnp.dot(x_ref[...], w_ref[0], preferred_element_type=jnp.float32)
        + b_ref[0]
    )


def _transform(x, w_all, b_all):
    return pl.pallas_call(
        _transform_body,
        grid=(N_NODES // _BM, N_REL1),
        in_specs=[
            pl.BlockSpec((_BM, IN_DIM), lambda j, r: (j, 0)),
            pl.BlockSpec((1, IN_DIM, OUT_DIM), lambda j, r: (r, 0, 0)),
            pl.BlockSpec((1, 1, OUT_DIM), lambda j, r: (r, 0, 0)),
        ],
        out_specs=pl.BlockSpec((1, _BM, OUT_DIM), lambda j, r: (r, j, 0)),
        out_shape=jax.ShapeDtypeStruct((N_REL1, N_NODES, OUT_DIM), jnp.float32),
    )(x, w_all, b_all)


_MESH = plsc.VectorSubcoreMesh(
    core_axis_name="c", subcore_axis_name="s", num_cores=NC, num_subcores=NS
)


@functools.partial(
    pl.kernel,
    out_type=jax.ShapeDtypeStruct((NC, ACC_ROWS, HALF), jnp.float32),
    mesh=_MESH,
    scratch_types=[
        pltpu.VMEM((NCH, CHUNK), jnp.int32),
        pltpu.VMEM((NCH, CHUNK), jnp.int32),
        pltpu.VMEM((CHUNK, HALF), jnp.float32),
        pltpu.VMEM((CHUNK, HALF), jnp.float32),
        pltpu.VMEM_SHARED((ACC_ROWS, HALF), jnp.float32),
        pltpu.SemaphoreType.DMA,
        pltpu.SemaphoreType.DMA,
    ],
    compiler_params=pltpu.CompilerParams(use_tc_tiling_on_sc=False),
)
def _edge_scatter(tbl_hbm, gidx_hbm, dst_hbm, out_hbm,
                  gidx_v, dst_v, rows_a, rows_b, acc_sh, sem_a, sem_b):
    c = lax.axis_index("c")
    s = lax.axis_index("s")

    @pl.loop(0, CHUNK)
    def _zrow(r):
        for u in range(HALF // LANES):
            rows_a[r, pl.ds(u * LANES, LANES)] = jnp.zeros(
                (LANES,), jnp.float32
            )

    for i in range(SPW // CHUNK):
        pltpu.sync_copy(rows_a, acc_sh.at[pl.ds(s * SPW + i * CHUNK, CHUNK)])
    _rem = SPW % CHUNK
    if _rem:
        pltpu.sync_copy(
            rows_a.at[pl.ds(0, _rem)],
            acc_sh.at[pl.ds(s * SPW + (SPW // CHUNK) * CHUNK, _rem)],
        )
    pltpu.sync_copy(gidx_hbm.at[s], gidx_v)
    pltpu.sync_copy(dst_hbm.at[s], dst_v)
    plsc.subcore_barrier()

    @pl.loop(0, NCH)
    def _xform(j):
        for u in range(CHUNK // LANES):
            g = gidx_v[j, pl.ds(u * LANES, LANES)]
            gidx_v[j, pl.ds(u * LANES, LANES)] = g * 2 + c

    def _gi(m):
        return gidx_v.at[m]

    def _di(m):
        return dst_v.at[m]

    pltpu.async_copy(tbl_hbm.at[_gi(0)], rows_a, sem_a)

    @pl.loop(0, (NDMA - 1) // 2)
    def _pairs(k):
        m = 2 * k
        pltpu.async_copy(tbl_hbm.at[_gi(m + 1)], rows_b, sem_b)
        pltpu.make_async_copy(tbl_hbm.at[_gi(m)], rows_a, sem_a).wait()
        pltpu.sync_copy(rows_a, acc_sh.at[_di(m)], add=True)
        pltpu.async_copy(tbl_hbm.at[_gi(m + 2)], rows_a, sem_a)
        pltpu.make_async_copy(tbl_hbm.at[_gi(m + 1)], rows_b, sem_b).wait()
        pltpu.sync_copy(rows_b, acc_sh.at[_di(m + 1)], add=True)

    pltpu.make_async_copy(tbl_hbm.at[_gi(NDMA - 1)], rows_a, sem_a).wait()
    pltpu.sync_copy(rows_a, acc_sh.at[_di(NDMA - 1)], add=True)

    plsc.subcore_barrier()
    pltpu.sync_copy(
        acc_sh.at[pl.ds(s * SPW, SPW)], out_hbm.at[c, pl.ds(s * SPW, SPW)]
    )


_BL = 10000


def _ln_body(t_ref, p_ref, w_ref, b_ref, o_ref):
    total = t_ref[0] + jnp.concatenate((p_ref[0], p_ref[1]), axis=-1)
    mu = jnp.mean(total, axis=1, keepdims=True)
    d = total - mu
    var = jnp.mean(d * d, axis=1, keepdims=True)
    o_ref[...] = d * lax.rsqrt(var + 1e-5) * w_ref[0] + b_ref[0]


def _layernorm(t, partials, ln_w, ln_b):
    return pl.pallas_call(
        _ln_body,
        grid=(N_NODES // _BL,),
        in_specs=[
            pl.BlockSpec((1, _BL, OUT_DIM), lambda j: (N_REL1 - 1, j, 0)),
            pl.BlockSpec((NC, _BL, HALF), lambda j: (0, j, 0)),
            pl.BlockSpec((1, OUT_DIM), lambda j: (0, 0)),
            pl.BlockSpec((1, OUT_DIM), lambda j: (0, 0)),
        ],
        out_specs=pl.BlockSpec((_BL, OUT_DIM), lambda j: (j, 0)),
        out_shape=jax.ShapeDtypeStruct((N_NODES, OUT_DIM), jnp.float32),
    )(t, partials, ln_w, ln_b)


def kernel(x, edge_index, edge_type, relation_weights, relation_bias,
           self_loop_w, self_loop_b, ln_weight, ln_bias):
    x = x.astype(jnp.float32)
    src = edge_index[0].astype(jnp.int32)
    dst = edge_index[1].astype(jnp.int32)
    et = edge_type.astype(jnp.int32)

    w_all = jnp.concatenate([relation_weights, self_loop_w.T[None]], axis=0)
    b_all = jnp.concatenate(
        [relation_bias, self_loop_b[None]], axis=0
    ).reshape(N_REL1, 1, OUT_DIM)

    t = _transform(x, w_all, b_all)
    tbl = t.reshape(N_REL1 * N_NODES * 2, HALF)

    pad = E_PAD - N_EDGES
    gidx = jnp.concatenate(
        [et * N_NODES + src, jnp.zeros((pad,), jnp.int32)]
    ).reshape(NS, NCH, CHUNK)
    dstp = jnp.concatenate(
        [dst, jnp.full((pad,), N_NODES, jnp.int32)]
    ).reshape(NS, NCH, CHUNK)

    partials = _edge_scatter(tbl, gidx, dstp)

    return _layernorm(
        t, partials, ln_weight.reshape(1, OUT_DIM), ln_bias.reshape(1, OUT_DIM)
    )
